# baseline (device time: 165556 ns/iter reference)
import jax
import jax.numpy as jnp
from jax import lax
from jax.experimental import pallas as pl
from jax.experimental.pallas import tpu as pltpu

N_DEV = 4


def kernel(x, w_mat, scale_x, scale_w):
    m_total, k_shard = x.shape
    k2, n = w_mat.shape
    assert k2 == k_shard
    m_per = m_total // N_DEV

    xb = x.astype(jnp.bfloat16)
    wb = w_mat.astype(jnp.bfloat16)

    def body(x_ref, w_ref, sx_ref, sw_ref, out_ref,
             comm_ref, send_sems, recv_sems):
        my = lax.axis_index("i")
        left = lax.rem(my + N_DEV - 1, N_DEV)
        right = lax.rem(my + 1, N_DEV)

        barrier_sem = pltpu.get_barrier_semaphore()
        pl.semaphore_signal(barrier_sem, inc=1, device_id=(left,),
                            device_id_type=pl.DeviceIdType.MESH)
        pl.semaphore_signal(barrier_sem, inc=1, device_id=(right,),
                            device_id_type=pl.DeviceIdType.MESH)
        pl.semaphore_wait(barrier_sem, 2)

        def partial(c):
            xs = x_ref[pl.ds(c * m_per, m_per), :]
            return jnp.dot(xs, w_ref[...], preferred_element_type=jnp.float32)

        c0 = lax.rem(my + N_DEV - 1, N_DEV)
        comm_ref[0] = partial(c0).astype(jnp.bfloat16)

        for h in range(N_DEV - 1):
            s_slot = h % 2
            r_slot = (h + 1) % 2
            rdma = pltpu.make_async_remote_copy(
                src_ref=comm_ref.at[s_slot],
                dst_ref=comm_ref.at[r_slot],
                send_sem=send_sems.at[s_slot],
                recv_sem=recv_sems.at[r_slot],
                device_id=(right,),
                device_id_type=pl.DeviceIdType.MESH,
            )
            rdma.start()
            c = lax.rem(my + N_DEV - 2 - h, N_DEV)
            p = partial(c)
            rdma.wait()
            if h < N_DEV - 2:
                acc = comm_ref[r_slot].astype(jnp.float32) + p
                comm_ref[r_slot] = acc.astype(jnp.bfloat16)
            else:
                s = sx_ref[0] * sw_ref[0]
                out_ref[...] = (comm_ref[r_slot].astype(jnp.float32) + p) * s

    return pl.pallas_call(
        body,
        out_shape=jax.ShapeDtypeStruct((m_per, n), jnp.float32),
        in_specs=[
            pl.BlockSpec(memory_space=pltpu.VMEM),
            pl.BlockSpec(memory_space=pltpu.VMEM),
            pl.BlockSpec(memory_space=pltpu.SMEM),
            pl.BlockSpec(memory_space=pltpu.SMEM),
        ],
        out_specs=pl.BlockSpec(memory_space=pltpu.VMEM),
        scratch_shapes=[
            pltpu.VMEM((2, m_per, n), jnp.bfloat16),
            pltpu.SemaphoreType.DMA((2,)),
            pltpu.SemaphoreType.DMA((2,)),
        ],
        compiler_params=pltpu.CompilerParams(collective_id=0),
    )(xb, wb, scale_x, scale_w)


# device time: 98144 ns/iter; 1.6869x vs baseline; 1.6869x over previous
import jax
import jax.numpy as jnp
from jax import lax
from jax.experimental import pallas as pl
from jax.experimental.pallas import tpu as pltpu

N_DEV = 4


def kernel(x, w_mat, scale_x, scale_w):
    m_total, k_shard = x.shape
    k2, n = w_mat.shape
    assert k2 == k_shard
    m_per = m_total // N_DEV
    n2 = n // 2

    xb = x.astype(jnp.bfloat16)
    wb = w_mat.astype(jnp.bfloat16)

    def body(x_ref, w_ref, sx_ref, sw_ref, out_ref,
             comm_l, comm_r, send_l, recv_l, send_r, recv_r):
        my = lax.axis_index("i")
        left = lax.rem(my + N_DEV - 1, N_DEV)
        right = lax.rem(my + 1, N_DEV)

        barrier_sem = pltpu.get_barrier_semaphore()
        pl.semaphore_signal(barrier_sem, inc=1, device_id=(left,),
                            device_id_type=pl.DeviceIdType.MESH)
        pl.semaphore_signal(barrier_sem, inc=1, device_id=(right,),
                            device_id_type=pl.DeviceIdType.MESH)
        pl.semaphore_wait(barrier_sem, 2)

        def p_l(c):
            xs = x_ref[pl.ds(c * m_per, m_per), :]
            return jnp.dot(xs, w_ref[:, :n2], preferred_element_type=jnp.float32)

        def p_r(c):
            xs = x_ref[pl.ds(c * m_per, m_per), :]
            return jnp.dot(xs, w_ref[:, n2:], preferred_element_type=jnp.float32)

        comm_l[0] = p_l(lax.rem(my + N_DEV - 1, N_DEV)).astype(jnp.bfloat16)
        comm_r[0] = p_r(lax.rem(my + 1, N_DEV)).astype(jnp.bfloat16)

        for h in range(N_DEV - 1):
            s_slot = h % 2
            r_slot = (h + 1) % 2
            rdma_cw = pltpu.make_async_remote_copy(
                src_ref=comm_l.at[s_slot],
                dst_ref=comm_l.at[r_slot],
                send_sem=send_l.at[s_slot],
                recv_sem=recv_l.at[r_slot],
                device_id=(right,),
                device_id_type=pl.DeviceIdType.MESH,
            )
            rdma_ccw = pltpu.make_async_remote_copy(
                src_ref=comm_r.at[s_slot],
                dst_ref=comm_r.at[r_slot],
                send_sem=send_r.at[s_slot],
                recv_sem=recv_r.at[r_slot],
                device_id=(left,),
                device_id_type=pl.DeviceIdType.MESH,
            )
            rdma_cw.start()
            rdma_ccw.start()
            cl = lax.rem(my + N_DEV - 2 - h, N_DEV)
            cr = lax.rem(my + 2 + h, N_DEV)
            pl_val = p_l(cl)
            pr_val = p_r(cr)
            rdma_cw.wait()
            rdma_ccw.wait()
            if h < N_DEV - 2:
                comm_l[r_slot] = (
                    comm_l[r_slot].astype(jnp.float32) + pl_val
                ).astype(jnp.bfloat16)
                comm_r[r_slot] = (
                    comm_r[r_slot].astype(jnp.float32) + pr_val
                ).astype(jnp.bfloat16)
            else:
                s = sx_ref[0] * sw_ref[0]
                out_ref[:, :n2] = (comm_l[r_slot].astype(jnp.float32) + pl_val) * s
                out_ref[:, n2:] = (comm_r[r_slot].astype(jnp.float32) + pr_val) * s

    return pl.pallas_call(
        body,
        out_shape=jax.ShapeDtypeStruct((m_per, n), jnp.float32),
        in_specs=[
            pl.BlockSpec(memory_space=pltpu.VMEM),
            pl.BlockSpec(memory_space=pltpu.VMEM),
            pl.BlockSpec(memory_space=pltpu.SMEM),
            pl.BlockSpec(memory_space=pltpu.SMEM),
        ],
        out_specs=pl.BlockSpec(memory_space=pltpu.VMEM),
        scratch_shapes=[
            pltpu.VMEM((2, m_per, n2), jnp.bfloat16),
            pltpu.VMEM((2, m_per, n2), jnp.bfloat16),
            pltpu.SemaphoreType.DMA((2,)),
            pltpu.SemaphoreType.DMA((2,)),
            pltpu.SemaphoreType.DMA((2,)),
            pltpu.SemaphoreType.DMA((2,)),
        ],
        compiler_params=pltpu.CompilerParams(collective_id=0),
    )(xb, wb, scale_x, scale_w)


# device time: 87831 ns/iter; 1.8849x vs baseline; 1.1174x over previous
import jax
import jax.numpy as jnp
from jax import lax
from jax.experimental import pallas as pl
from jax.experimental.pallas import tpu as pltpu

N_DEV = 4
N_STRIP = 2


def kernel(x, w_mat, scale_x, scale_w):
    m_total, k_shard = x.shape
    k2, n = w_mat.shape
    assert k2 == k_shard
    m_per = m_total // N_DEV
    n2 = n // 2
    ns = n2 // N_STRIP

    xb = x.astype(jnp.bfloat16)
    wb = w_mat.astype(jnp.bfloat16)

    def body(x_ref, w_ref, sx_ref, sw_ref, out_ref,
             comm_l, comm_r, send_l, recv_l, send_r, recv_r):
        my = lax.axis_index("i")
        left = lax.rem(my + N_DEV - 1, N_DEV)
        right = lax.rem(my + 1, N_DEV)

        barrier_sem = pltpu.get_barrier_semaphore()
        pl.semaphore_signal(barrier_sem, inc=1, device_id=(left,),
                            device_id_type=pl.DeviceIdType.MESH)
        pl.semaphore_signal(barrier_sem, inc=1, device_id=(right,),
                            device_id_type=pl.DeviceIdType.MESH)
        pl.semaphore_wait(barrier_sem, 2)

        def xs(c):
            return x_ref[pl.ds(c * m_per, m_per), :]

        def p_l(c):
            return jnp.dot(xs(c), w_ref[:, :n2],
                           preferred_element_type=jnp.float32)

        def p_r(c):
            return jnp.dot(xs(c), w_ref[:, n2:],
                           preferred_element_type=jnp.float32)

        def cw_rdma(h, s):
            return pltpu.make_async_remote_copy(
                src_ref=comm_l.at[h % 2, :, s * ns:(s + 1) * ns],
                dst_ref=comm_l.at[(h + 1) % 2, :, s * ns:(s + 1) * ns],
                send_sem=send_l.at[h % 2, s],
                recv_sem=recv_l.at[(h + 1) % 2, s],
                device_id=(right,),
                device_id_type=pl.DeviceIdType.MESH,
            )

        def ccw_rdma(h, s):
            return pltpu.make_async_remote_copy(
                src_ref=comm_r.at[h % 2, :, s * ns:(s + 1) * ns],
                dst_ref=comm_r.at[(h + 1) % 2, :, s * ns:(s + 1) * ns],
                send_sem=send_r.at[h % 2, s],
                recv_sem=recv_r.at[(h + 1) % 2, s],
                device_id=(left,),
                device_id_type=pl.DeviceIdType.MESH,
            )

        cl0 = lax.rem(my + N_DEV - 1, N_DEV)
        cr0 = lax.rem(my + 1, N_DEV)
        for s in range(N_STRIP):
            sl = slice(s * ns, (s + 1) * ns)
            comm_l[0, :, sl] = jnp.dot(
                xs(cl0), w_ref[:, sl], preferred_element_type=jnp.float32
            ).astype(jnp.bfloat16)
            cw_rdma(0, s).start()
            comm_r[0, :, sl] = jnp.dot(
                xs(cr0), w_ref[:, n2 + s * ns:n2 + (s + 1) * ns],
                preferred_element_type=jnp.float32
            ).astype(jnp.bfloat16)
            ccw_rdma(0, s).start()

        for h in range(N_DEV - 1):
            r_slot = (h + 1) % 2
            cl = lax.rem(my + N_DEV - 2 - h, N_DEV)
            cr = lax.rem(my + 2 + h, N_DEV)
            pl_val = p_l(cl)
            pr_val = p_r(cr)
            last = h == N_DEV - 2
            if last:
                scale = sx_ref[0] * sw_ref[0]
            for s in range(N_STRIP):
                sl = slice(s * ns, (s + 1) * ns)
                cw_rdma(h, s).wait()
                if not last:
                    comm_l[r_slot, :, sl] = (
                        comm_l[r_slot, :, sl].astype(jnp.float32)
                        + pl_val[:, sl]
                    ).astype(jnp.bfloat16)
                    cw_rdma(h + 1, s).start()
                else:
                    out_ref[:, sl] = (
                        comm_l[r_slot, :, sl].astype(jnp.float32)
                        + pl_val[:, sl]
                    ) * scale
                ccw_rdma(h, s).wait()
                if not last:
                    comm_r[r_slot, :, sl] = (
                        comm_r[r_slot, :, sl].astype(jnp.float32)
                        + pr_val[:, sl]
                    ).astype(jnp.bfloat16)
                    ccw_rdma(h + 1, s).start()
                else:
                    out_ref[:, n2 + s * ns:n2 + (s + 1) * ns] = (
                        comm_r[r_slot, :, sl].astype(jnp.float32)
                        + pr_val[:, sl]
                    ) * scale

    return pl.pallas_call(
        body,
        out_shape=jax.ShapeDtypeStruct((m_per, n), jnp.float32),
        in_specs=[
            pl.BlockSpec(memory_space=pltpu.VMEM),
            pl.BlockSpec(memory_space=pltpu.VMEM),
            pl.BlockSpec(memory_space=pltpu.SMEM),
            pl.BlockSpec(memory_space=pltpu.SMEM),
        ],
        out_specs=pl.BlockSpec(memory_space=pltpu.VMEM),
        scratch_shapes=[
            pltpu.VMEM((2, m_per, n2), jnp.bfloat16),
            pltpu.VMEM((2, m_per, n2), jnp.bfloat16),
            pltpu.SemaphoreType.DMA((2, N_STRIP)),
            pltpu.SemaphoreType.DMA((2, N_STRIP)),
            pltpu.SemaphoreType.DMA((2, N_STRIP)),
            pltpu.SemaphoreType.DMA((2, N_STRIP)),
        ],
        compiler_params=pltpu.CompilerParams(collective_id=0),
    )(xb, wb, scale_x, scale_w)


# device time: 83531 ns/iter; 1.9820x vs baseline; 1.0515x over previous
import jax
import jax.numpy as jnp
from jax import lax
from jax.experimental import pallas as pl
from jax.experimental.pallas import tpu as pltpu

N_DEV = 4
N_STRIP = 2


def kernel(x, w_mat, scale_x, scale_w):
    m_total, k_shard = x.shape
    k2, n = w_mat.shape
    assert k2 == k_shard
    m_per = m_total // N_DEV
    n2 = n // 2
    ns = n2 // N_STRIP

    wb = w_mat.astype(jnp.bfloat16)

    def body(x_ref, w_ref, sx_ref, sw_ref, out_ref,
             comm_l, comm_r, x_stage, send_l, recv_l, send_r, recv_r,
             stage_sems):
        my = lax.axis_index("i")
        left = lax.rem(my + N_DEV - 1, N_DEV)
        right = lax.rem(my + 1, N_DEV)

        def stage(c, slot):
            return pltpu.make_async_copy(
                x_ref.at[pl.ds(c * m_per, m_per), :],
                x_stage.at[slot],
                stage_sems.at[slot],
            )

        c_m1 = lax.rem(my + N_DEV - 1, N_DEV)
        c_p1 = lax.rem(my + 1, N_DEV)
        c_p2 = lax.rem(my + 2, N_DEV)
        s0 = stage(c_m1, 0)
        s0.start()
        s1 = stage(c_p1, 1)
        s1.start()

        barrier_sem = pltpu.get_barrier_semaphore()
        pl.semaphore_signal(barrier_sem, inc=1, device_id=(left,),
                            device_id_type=pl.DeviceIdType.MESH)
        pl.semaphore_signal(barrier_sem, inc=1, device_id=(right,),
                            device_id_type=pl.DeviceIdType.MESH)
        pl.semaphore_wait(barrier_sem, 2)

        def cw_rdma(h, s):
            return pltpu.make_async_remote_copy(
                src_ref=comm_l.at[h % 2, :, s * ns:(s + 1) * ns],
                dst_ref=comm_l.at[(h + 1) % 2, :, s * ns:(s + 1) * ns],
                send_sem=send_l.at[h % 2, s],
                recv_sem=recv_l.at[(h + 1) % 2, s],
                device_id=(right,),
                device_id_type=pl.DeviceIdType.MESH,
            )

        def ccw_rdma(h, s):
            return pltpu.make_async_remote_copy(
                src_ref=comm_r.at[h % 2, :, s * ns:(s + 1) * ns],
                dst_ref=comm_r.at[(h + 1) % 2, :, s * ns:(s + 1) * ns],
                send_sem=send_r.at[h % 2, s],
                recv_sem=recv_r.at[(h + 1) % 2, s],
                device_id=(left,),
                device_id_type=pl.DeviceIdType.MESH,
            )

        s0.wait()
        xb0 = x_stage[0].astype(jnp.bfloat16)
        comm_l[0, :, 0:ns] = jnp.dot(
            xb0, w_ref[:, 0:ns], preferred_element_type=jnp.float32
        ).astype(jnp.bfloat16)
        cw_rdma(0, 0).start()
        s1.wait()
        xb1 = x_stage[1].astype(jnp.bfloat16)
        comm_r[0, :, 0:ns] = jnp.dot(
            xb1, w_ref[:, n2:n2 + ns], preferred_element_type=jnp.float32
        ).astype(jnp.bfloat16)
        ccw_rdma(0, 0).start()
        comm_l[0, :, ns:n2] = jnp.dot(
            xb0, w_ref[:, ns:n2], preferred_element_type=jnp.float32
        ).astype(jnp.bfloat16)
        cw_rdma(0, 1).start()
        comm_r[0, :, ns:n2] = jnp.dot(
            xb1, w_ref[:, n2 + ns:], preferred_element_type=jnp.float32
        ).astype(jnp.bfloat16)
        ccw_rdma(0, 1).start()

        s2 = stage(c_p2, 0)
        s2.start()

        for h in range(N_DEV - 1):
            r_slot = (h + 1) % 2
            if h == 0:
                s2.wait()
                xc = x_stage[0].astype(jnp.bfloat16)
                pl_val = jnp.dot(xc, w_ref[:, :n2],
                                 preferred_element_type=jnp.float32)
                pr_val = jnp.dot(xc, w_ref[:, n2:],
                                 preferred_element_type=jnp.float32)
                s3 = stage(c_m1, 0)
                s3.start()
            elif h == 1:
                pl_val = jnp.dot(x_stage[1].astype(jnp.bfloat16),
                                 w_ref[:, :n2],
                                 preferred_element_type=jnp.float32)
                s3.wait()
                pr_val = jnp.dot(x_stage[0].astype(jnp.bfloat16),
                                 w_ref[:, n2:],
                                 preferred_element_type=jnp.float32)
                s4 = stage(my, 1)
                s4.start()
            else:
                s4.wait()
                xc = x_stage[1].astype(jnp.bfloat16)
                pl_val = jnp.dot(xc, w_ref[:, :n2],
                                 preferred_element_type=jnp.float32)
                pr_val = jnp.dot(xc, w_ref[:, n2:],
                                 preferred_element_type=jnp.float32)
            last = h == N_DEV - 2
            if last:
                scale = sx_ref[0] * sw_ref[0]
            for s in range(N_STRIP):
                sl = slice(s * ns, (s + 1) * ns)
                cw_rdma(h, s).wait()
                if not last:
                    comm_l[r_slot, :, sl] = (
                        comm_l[r_slot, :, sl].astype(jnp.float32)
                        + pl_val[:, sl]
                    ).astype(jnp.bfloat16)
                    cw_rdma(h + 1, s).start()
                else:
                    out_ref[:, sl] = (
                        comm_l[r_slot, :, sl].astype(jnp.float32)
                        + pl_val[:, sl]
                    ) * scale
                ccw_rdma(h, s).wait()
                if not last:
                    comm_r[r_slot, :, sl] = (
                        comm_r[r_slot, :, sl].astype(jnp.float32)
                        + pr_val[:, sl]
                    ).astype(jnp.bfloat16)
                    ccw_rdma(h + 1, s).start()
                else:
                    out_ref[:, n2 + s * ns:n2 + (s + 1) * ns] = (
                        comm_r[r_slot, :, sl].astype(jnp.float32)
                        + pr_val[:, sl]
                    ) * scale

    return pl.pallas_call(
        body,
        out_shape=jax.ShapeDtypeStruct((m_per, n), jnp.float32),
        in_specs=[
            pl.BlockSpec(memory_space=pl.ANY),
            pl.BlockSpec(memory_space=pltpu.VMEM),
            pl.BlockSpec(memory_space=pltpu.SMEM),
            pl.BlockSpec(memory_space=pltpu.SMEM),
        ],
        out_specs=pl.BlockSpec(memory_space=pltpu.VMEM),
        scratch_shapes=[
            pltpu.VMEM((2, m_per, n2), jnp.bfloat16),
            pltpu.VMEM((2, m_per, n2), jnp.bfloat16),
            pltpu.VMEM((2, m_per, k_shard), jnp.float32),
            pltpu.SemaphoreType.DMA((2, N_STRIP)),
            pltpu.SemaphoreType.DMA((2, N_STRIP)),
            pltpu.SemaphoreType.DMA((2, N_STRIP)),
            pltpu.SemaphoreType.DMA((2, N_STRIP)),
            pltpu.SemaphoreType.DMA((2,)),
        ],
        compiler_params=pltpu.CompilerParams(collective_id=0),
    )(x, wb, scale_x, scale_w)
